# baseline (speedup 1.0000x reference)
import functools

import jax
import jax.numpy as jnp
from jax import lax
from jax.experimental import pallas as pl
from jax.experimental.pallas import tpu as pltpu
from jax.experimental.pallas import tpu_sc as plsc

N = 10000
F = 128
G = 128
NC = 2
NS = 16
NW = NC * NS
C = 128
NP = 10112
SLAB = NP // NS
EC = 3 * C

MCK_L = 80
MCK_S = 40
NKP_L = NW * MCK_L + 4
NKP_S = NW * MCK_S + 4

_mesh = plsc.VectorSubcoreMesh(core_axis_name="c", subcore_axis_name="s")
_sc_params = pltpu.CompilerParams(needs_layout_passes=False)


@functools.partial(
    pl.kernel,
    out_type=(
        jax.ShapeDtypeStruct((NC * NP,), jnp.float32),
        jax.ShapeDtypeStruct((NC * NP,), jnp.float32),
    ),
    mesh=_mesh,
    compiler_params=_sc_params,
    scratch_types=[
        pltpu.VMEM_SHARED((NP,), jnp.float32),
        pltpu.VMEM_SHARED((NP,), jnp.float32),
        pltpu.VMEM((4 * EC,), jnp.int32),
        pltpu.VMEM((C,), jnp.int32),
        pltpu.VMEM((C,), jnp.float32),
        pltpu.SemaphoreType.DMA,
        pltpu.SemaphoreType.DMA,
        pltpu.SemaphoreType.DMA,
        pltpu.SemaphoreType.DMA,
    ],
)
def _deg_kernel(ed_l, ed_s, z1, degp_l, degp_s,
                acc_l, acc_s, ebuf, dstb, ewb, es0, es1, es2, es3):
    c = lax.axis_index("c")
    s = lax.axis_index("s")
    w = s * NC + c
    esems = (es0, es1, es2, es3)

    @pl.when(s == 0)
    def _():
        pltpu.sync_copy(z1, acc_l)
        pltpu.sync_copy(z1, acc_s)

    plsc.subcore_barrier()

    def run(ed_hbm, mck, acc):
        base = w * mck
        for r in range(4):
            pltpu.async_copy(ed_hbm.at[pl.ds((base + r) * EC, EC)],
                             ebuf.at[pl.ds(r * EC, EC)], esems[r])

        def sub(jj, r):
            pltpu.make_async_copy(ed_hbm.at[pl.ds(0, EC)],
                                  ebuf.at[pl.ds(r * EC, EC)],
                                  esems[r]).wait()
            for t in range(C // 16):
                dstb[pl.ds(t * 16, 16)] = ebuf[pl.ds(r * EC + C + t * 16, 16)]
                ewb[pl.ds(t * 16, 16)] = lax.bitcast_convert_type(
                    ebuf[pl.ds(r * EC + 2 * C + t * 16, 16)], jnp.float32)
            pltpu.sync_copy(ewb, acc.at[dstb], add=True)
            pltpu.async_copy(ed_hbm.at[pl.ds((base + jj + 4) * EC, EC)],
                             ebuf.at[pl.ds(r * EC, EC)], esems[r])

        def body(i, _):
            sub(4 * i, 0)
            sub(4 * i + 1, 1)
            sub(4 * i + 2, 2)
            sub(4 * i + 3, 3)
            return 0

        lax.fori_loop(0, mck // 4, body, 0)
        for r in range(4):
            pltpu.make_async_copy(ed_hbm.at[pl.ds(0, EC)],
                                  ebuf.at[pl.ds(r * EC, EC)],
                                  esems[r]).wait()

    run(ed_l, MCK_L, acc_l)
    run(ed_s, MCK_S, acc_s)
    plsc.subcore_barrier()

    @pl.when(s == 0)
    def _():
        pltpu.sync_copy(acc_l, degp_l.at[pl.ds(c * NP, NP)])
        pltpu.sync_copy(acc_s, degp_s.at[pl.ds(c * NP, NP)])


def _mm_body(xl_ref, wa_ref, xs_ref, wb_ref, hl_ref, hs_ref):
    hl_ref[...] = jnp.dot(xl_ref[...], wa_ref[...],
                          preferred_element_type=jnp.float32)
    hs_ref[...] = jnp.dot(xs_ref[...], wb_ref[...],
                          preferred_element_type=jnp.float32)


def _dinv_body(dl_ref, ds_ref, ol_ref, os_ref):
    degl = dl_ref[0] + dl_ref[1]
    ol_ref[...] = jnp.where(degl > 0, lax.rsqrt(degl), 0.0)
    degs = ds_ref[0] + ds_ref[1]
    os_ref[...] = jnp.where(degs > 0, lax.rsqrt(degs), 0.0)


@functools.partial(
    pl.kernel,
    out_type=(
        jax.ShapeDtypeStruct((NC, NP, F), jnp.float32),
        jax.ShapeDtypeStruct((NC, NP, F), jnp.float32),
    ),
    mesh=_mesh,
    compiler_params=_sc_params,
    scratch_types=[
        pltpu.VMEM_SHARED((NP, F), jnp.float32),
        pltpu.VMEM((4 * EC,), jnp.int32),
        pltpu.VMEM((C, F), jnp.float32),
        pltpu.VMEM((C, F), jnp.float32),
        pltpu.VMEM((C,), jnp.float32),
        pltpu.VMEM((C,), jnp.float32),
        pltpu.VMEM((C,), jnp.int32),
        pltpu.VMEM((C,), jnp.int32),
        pltpu.VMEM((C,), jnp.int32),
        pltpu.SemaphoreType.DMA,
        pltpu.SemaphoreType.DMA,
        pltpu.SemaphoreType.DMA,
        pltpu.SemaphoreType.DMA,
        pltpu.SemaphoreType.DMA,
        pltpu.SemaphoreType.DMA,
        pltpu.SemaphoreType.DMA,
        pltpu.SemaphoreType.DMA,
    ],
)
def _msg_kernel(h_l, h_s, dinv_l, dinv_s, ed_l, ed_s, z128,
                accp_l, accp_s,
                acc, ebuf, rows0, rows1, dinvb0, dinvb1, srcb0, srcb1, dstb,
                es0, es1, es2, es3, gs0, gs1, ds0, ds1):
    c = lax.axis_index("c")
    s = lax.axis_index("s")
    w = s * NC + c
    esems = (es0, es1, es2, es3)

    def graph_pass(h_hbm, dinv_hbm, ed_hbm, mck, out_hbm):
        pltpu.sync_copy(z128.at[pl.ds(s * SLAB, SLAB)],
                        acc.at[pl.ds(s * SLAB, SLAB)])
        base = w * mck
        for r in range(4):
            pltpu.async_copy(ed_hbm.at[pl.ds((base + r) * EC, EC)],
                             ebuf.at[pl.ds(r * EC, EC)], esems[r])
        for r, srcb, rows, dinvb, gsem, dsem in (
                (0, srcb0, rows0, dinvb0, gs0, ds0),
                (1, srcb1, rows1, dinvb1, gs1, ds1)):
            pltpu.make_async_copy(ed_hbm.at[pl.ds(0, EC)],
                                  ebuf.at[pl.ds(r * EC, EC)],
                                  esems[r]).wait()
            for t in range(C // 16):
                srcb[pl.ds(t * 16, 16)] = ebuf[pl.ds(r * EC + t * 16, 16)]
            pltpu.async_copy(h_hbm.at[srcb], rows, gsem)
            pltpu.async_copy(dinv_hbm.at[srcb], dinvb, dsem)
        plsc.subcore_barrier()

        def sub(jj, r, r2, rows, dinvb, srcb, gsem, dsem):
            pltpu.make_async_copy(dinv_hbm.at[srcb], dinvb, dsem).wait()
            pltpu.make_async_copy(h_hbm.at[srcb], rows, gsem).wait()
            for t in range(C // 16):
                ewv = lax.bitcast_convert_type(
                    ebuf[pl.ds(r * EC + 2 * C + t * 16, 16)], jnp.float32)
                dinvb[pl.ds(t * 16, 16)] = dinvb[pl.ds(t * 16, 16)] * ewv
                dstb[pl.ds(t * 16, 16)] = ebuf[pl.ds(r * EC + C + t * 16, 16)]

            def scale(e, _):
                nspl = plsc.load_gather(dinvb, [lax.broadcast(e, (16,))])
                for f in range(F // 16):
                    rows[e, pl.ds(f * 16, 16)] = (
                        rows[e, pl.ds(f * 16, 16)] * nspl)
                return 0

            lax.fori_loop(0, C, scale, 0)
            pltpu.sync_copy(rows, acc.at[dstb], add=True)
            pltpu.make_async_copy(ed_hbm.at[pl.ds(0, EC)],
                                  ebuf.at[pl.ds(r2 * EC, EC)],
                                  esems[r2]).wait()
            for t in range(C // 16):
                srcb[pl.ds(t * 16, 16)] = ebuf[pl.ds(r2 * EC + t * 16, 16)]
            pltpu.async_copy(h_hbm.at[srcb], rows, gsem)
            pltpu.async_copy(dinv_hbm.at[srcb], dinvb, dsem)
            pltpu.async_copy(ed_hbm.at[pl.ds((base + jj + 4) * EC, EC)],
                             ebuf.at[pl.ds(r * EC, EC)], esems[r])

        def body(i, _):
            sub(4 * i, 0, 2, rows0, dinvb0, srcb0, gs0, ds0)
            sub(4 * i + 1, 1, 3, rows1, dinvb1, srcb1, gs1, ds1)
            sub(4 * i + 2, 2, 0, rows0, dinvb0, srcb0, gs0, ds0)
            sub(4 * i + 3, 3, 1, rows1, dinvb1, srcb1, gs1, ds1)
            return 0

        lax.fori_loop(0, mck // 4, body, 0)
        for r, srcb, rows, dinvb, gsem, dsem in (
                (2, srcb0, rows0, dinvb0, gs0, ds0),
                (3, srcb1, rows1, dinvb1, gs1, ds1)):
            pltpu.make_async_copy(ed_hbm.at[pl.ds(0, EC)],
                                  ebuf.at[pl.ds(r * EC, EC)],
                                  esems[r]).wait()
            pltpu.make_async_copy(h_hbm.at[srcb], rows, gsem).wait()
            pltpu.make_async_copy(dinv_hbm.at[srcb], dinvb, dsem).wait()
        plsc.subcore_barrier()
        pltpu.sync_copy(acc.at[pl.ds(s * SLAB, SLAB)],
                        out_hbm.at[c, pl.ds(s * SLAB, SLAB)])
        plsc.subcore_barrier()

    graph_pass(h_l, dinv_l, ed_l, MCK_L, accp_l)
    graph_pass(h_s, dinv_s, ed_s, MCK_S, accp_s)


def _head_body(accp_l_ref, accp_s_ref, degp_l_ref, degp_s_ref,
               batch_l_ref, batch_s_ref, b1a_ref, b1b_ref,
               wl1_ref, bl1_ref, wl2_ref, bl2_ref,
               gamma_ref, beta_ref, wo_ref, bo_ref,
               out_ref, hidden_ref):
    def pooled(accp_ref, degp_ref, batch_ref, b_ref):
        acc = accp_ref[0] + accp_ref[1]
        deg = degp_ref[0] + degp_ref[1]
        dinv = jnp.where(deg > 0, lax.rsqrt(deg), 0.0)
        node = jnp.maximum(acc * dinv + b_ref[...], 0.0)
        iota = lax.broadcasted_iota(jnp.int32, (G, NP), 0)
        pt = (iota == batch_ref[...]).astype(jnp.float32)
        seg = jnp.dot(pt, node, preferred_element_type=jnp.float32,
                      precision=lax.Precision.HIGHEST)
        cnt = jnp.sum(pt, axis=1, keepdims=True)
        return seg / jnp.maximum(cnt, 1.0)

    h1 = pooled(accp_l_ref, degp_l_ref, batch_l_ref, b1a_ref)
    h2 = pooled(accp_s_ref, degp_s_ref, batch_s_ref, b1b_ref)
    hid = jnp.concatenate([h1, h2], axis=1)
    hid = jnp.dot(hid, wl1_ref[...],
                  preferred_element_type=jnp.float32) + bl1_ref[...]
    hid = jnp.dot(hid, wl2_ref[...],
                  preferred_element_type=jnp.float32) + bl2_ref[...]
    mean = jnp.mean(hid, axis=0, keepdims=True)
    var = jnp.mean((hid - mean) ** 2, axis=0, keepdims=True)
    hid = gamma_ref[...] * (hid - mean) / jnp.sqrt(var + 1e-5) + beta_ref[...]
    hid = jnp.maximum(hid, 0.0)
    hidden_ref[...] = hid
    out_ref[...] = jnp.dot(hid, wo_ref[...],
                           preferred_element_type=jnp.float32) + bo_ref[...]


def _pad_deg(degp_flat):
    return degp_flat.reshape(NC, NP, 1)


def _pad_batch(batch):
    return jnp.pad(batch, (0, NP - N), constant_values=-1).reshape(1, NP)


def _pack_edges(edge_index, edge_weight, nkpad):
    nk = edge_weight.shape[0] // C

    def chunk(a):
        return jnp.pad(a.reshape(nk, 1, C), ((0, nkpad - nk), (0, 0), (0, 0)))

    ed = jnp.concatenate(
        [chunk(edge_index[0]), chunk(edge_index[1]),
         chunk(lax.bitcast_convert_type(edge_weight, jnp.int32))], axis=1)
    return ed.reshape(-1)


def kernel(x_l, edge_index_l, edge_weight_l, x_s, edge_index_s, edge_weight_s,
           batch_index_l, batch_index_s,
           W1a, b1a, W1b, b1b, Wl1, bl1, Wl2, bl2, gamma, beta, Wo, bo):
    ed_l = _pack_edges(edge_index_l, edge_weight_l, NKP_L)
    ed_s = _pack_edges(edge_index_s, edge_weight_s, NKP_S)
    z1 = jnp.zeros((NP,), jnp.float32)
    z128 = jnp.zeros((NP, F), jnp.float32)

    degp_l, degp_s = _deg_kernel(ed_l, ed_s, z1)

    h_l, h_s = pl.pallas_call(
        _mm_body,
        out_shape=(jax.ShapeDtypeStruct((N, F), jnp.float32),
                   jax.ShapeDtypeStruct((N, F), jnp.float32)),
    )(x_l, W1a, x_s, W1b)

    dinv_l, dinv_s = pl.pallas_call(
        _dinv_body,
        out_shape=(jax.ShapeDtypeStruct((NP // 128, 128), jnp.float32),
                   jax.ShapeDtypeStruct((NP // 128, 128), jnp.float32)),
    )(degp_l.reshape(NC, NP // 128, 128), degp_s.reshape(NC, NP // 128, 128))

    accp_l, accp_s = _msg_kernel(h_l, h_s,
                                 dinv_l.reshape(NP), dinv_s.reshape(NP),
                                 ed_l, ed_s, z128)

    out, hidden = pl.pallas_call(
        _head_body,
        out_shape=(jax.ShapeDtypeStruct((G, 1), jnp.float32),
                   jax.ShapeDtypeStruct((G, F), jnp.float32)),
    )(accp_l, accp_s,
      _pad_deg(degp_l), _pad_deg(degp_s),
      _pad_batch(batch_index_l), _pad_batch(batch_index_s),
      b1a.reshape(1, F), b1b.reshape(1, F),
      Wl1, bl1.reshape(1, 2 * F), Wl2, bl2.reshape(1, F),
      gamma.reshape(1, F), beta.reshape(1, F), Wo, bo.reshape(1, 1))
    return (out, hidden)
